# Initial kernel scaffold; baseline (speedup 1.0000x reference)
import functools

import jax
import jax.numpy as jnp
from jax import lax
from jax.experimental import pallas as pl
from jax.experimental.pallas import tpu as pltpu
from jax.experimental.pallas import tpu_sc as plsc


_ROWS_PER_DMA = 128


def _sc_gather(word_emb, tok2d, n_chunks):
    D = word_emb.shape[1]
    info = plsc.get_sparse_core_info()
    NC, NS = info.num_cores, info.num_subcores
    NW = NC * NS
    total_rows = tok2d.shape[0] * _ROWS_PER_DMA
    mesh = plsc.VectorSubcoreMesh(core_axis_name="c", subcore_axis_name="s")

    @functools.partial(
        pl.kernel,
        out_type=jax.ShapeDtypeStruct((total_rows, D), jnp.float32),
        mesh=mesh,
        scratch_types=[
            pltpu.VMEM((n_chunks, _ROWS_PER_DMA), jnp.int32),
            pltpu.VMEM((_ROWS_PER_DMA, D), jnp.float32),
            pltpu.VMEM((_ROWS_PER_DMA, D), jnp.float32),
            pltpu.SemaphoreType.DMA,
            pltpu.SemaphoreType.DMA,
        ],
    )
    def k(table_hbm, idx_hbm, out_hbm, idx_v, rows0, rows1, sem0, sem1):
        wid = lax.axis_index("s") * NC + lax.axis_index("c")
        ibase = wid * n_chunks
        pltpu.sync_copy(idx_hbm.at[pl.ds(ibase, n_chunks)], idx_v)
        rows = (rows0, rows1)
        sems = (sem0, sem1)
        pltpu.async_copy(table_hbm.at[idx_v.at[0]], rows0, sem0)

        def chunk(j, _):
            @pl.when(j + 1 < n_chunks)
            def _():
                for b in range(2):
                    @pl.when(lax.rem(j + 1, 2) == b)
                    def _():
                        pltpu.async_copy(
                            table_hbm.at[idx_v.at[j + 1]], rows[b], sems[b])

            for b in range(2):
                @pl.when(lax.rem(j, 2) == b)
                def _():
                    pltpu.make_async_copy(
                        table_hbm.at[idx_v.at[j]], rows[b], sems[b]).wait()
                    pltpu.sync_copy(
                        rows[b],
                        out_hbm.at[pl.ds((ibase + j) * _ROWS_PER_DMA,
                                         _ROWS_PER_DMA)])
            return 0

        lax.fori_loop(0, n_chunks, chunk, 0)

    return k(word_emb, tok2d)


_EPS = 1e-12


def _ln_body(g_ref, seg_ref, pe_ref, te_ref, gamma_ref, beta_ref, o_ref):
    pe = pe_ref[...]
    te0 = te_ref[0, :]
    te1 = te_ref[1, :]
    seg = seg_ref[...]
    te = jnp.where(seg[..., None] == 0, te0[None, None, :], te1[None, None, :])
    x = g_ref[...] + te + pe[None, :, :]
    mu = jnp.mean(x, axis=-1, keepdims=True)
    xc = x - mu
    var = jnp.mean(xc * xc, axis=-1, keepdims=True)
    normed = xc * lax.rsqrt(var + _EPS)
    o_ref[...] = normed * gamma_ref[0, :] + beta_ref[0, :]


def _tc_ln(gathered, segment_ids, pe, te, gamma, beta, b_blk):
    B, S, D = gathered.shape
    grid = (B // b_blk,)
    return pl.pallas_call(
        _ln_body,
        grid=grid,
        in_specs=[
            pl.BlockSpec((b_blk, S, D), lambda i: (i, 0, 0)),
            pl.BlockSpec((b_blk, S), lambda i: (i, 0)),
            pl.BlockSpec((S, D), lambda i: (0, 0)),
            pl.BlockSpec((2, D), lambda i: (0, 0)),
            pl.BlockSpec((1, D), lambda i: (0, 0)),
            pl.BlockSpec((1, D), lambda i: (0, 0)),
        ],
        out_specs=pl.BlockSpec((b_blk, S, D), lambda i: (i, 0, 0)),
        out_shape=jax.ShapeDtypeStruct((B, S, D), jnp.float32),
    )(gathered, segment_ids, pe, te, gamma, beta)


def kernel(token_ids, segment_ids, word_emb, pos_emb, type_emb, gamma, beta):
    B, S = token_ids.shape
    D = word_emb.shape[1]
    N = B * S
    NW = 32
    n_chunks = N // (NW * _ROWS_PER_DMA)
    tok2d = token_ids.reshape(N // _ROWS_PER_DMA, _ROWS_PER_DMA)
    tok2d = tok2d.astype(jnp.int32)
    gathered = _sc_gather(word_emb, tok2d, n_chunks)
    out = _tc_ln(
        gathered.reshape(B, S, D),
        segment_ids.astype(jnp.int32),
        pos_emb[:S],
        type_emb,
        gamma.reshape(1, D),
        beta.reshape(1, D),
        b_blk=8,
    )
    return out


# speedup vs baseline: 3.8760x; 3.8760x over previous
import functools

import jax
import jax.numpy as jnp
from jax import lax
from jax.experimental import pallas as pl
from jax.experimental.pallas import tpu as pltpu
from jax.experimental.pallas import tpu_sc as plsc


_ROWS_PER_DMA = 128


def _sc_gather(word_emb, tok_flat, n_chunks):
    D = word_emb.shape[1]
    info = plsc.get_sparse_core_info()
    NC, NS = info.num_cores, info.num_subcores
    NW = NC * NS
    N = tok_flat.shape[0]
    per_w = N // NW
    mesh = plsc.VectorSubcoreMesh(core_axis_name="c", subcore_axis_name="s")

    @functools.partial(
        pl.kernel,
        out_type=jax.ShapeDtypeStruct((N, D), jnp.float32),
        mesh=mesh,
        scratch_types=[
            pltpu.VMEM((per_w,), jnp.int32),
            pltpu.VMEM((_ROWS_PER_DMA, D), jnp.float32),
            pltpu.VMEM((_ROWS_PER_DMA, D), jnp.float32),
            pltpu.SemaphoreType.DMA,
            pltpu.SemaphoreType.DMA,
        ],
    )
    def k(table_hbm, idx_hbm, out_hbm, idx_v, rows0, rows1, sem0, sem1):
        wid = lax.axis_index("s") * NC + lax.axis_index("c")
        base = wid * per_w
        pltpu.sync_copy(idx_hbm.at[pl.ds(base, per_w)], idx_v)
        rows = (rows0, rows1)
        sems = (sem0, sem1)
        pltpu.async_copy(
            table_hbm.at[idx_v.at[pl.ds(0, _ROWS_PER_DMA)]], rows0, sem0)

        def chunk(j, _):
            @pl.when(j + 1 < n_chunks)
            def _():
                for b in range(2):
                    @pl.when(lax.rem(j + 1, 2) == b)
                    def _():
                        pltpu.async_copy(
                            table_hbm.at[
                                idx_v.at[pl.ds((j + 1) * _ROWS_PER_DMA,
                                               _ROWS_PER_DMA)]],
                            rows[b], sems[b])

            for b in range(2):
                @pl.when(lax.rem(j, 2) == b)
                def _():
                    pltpu.make_async_copy(
                        table_hbm.at[
                            idx_v.at[pl.ds(j * _ROWS_PER_DMA,
                                           _ROWS_PER_DMA)]],
                        rows[b], sems[b]).wait()
                    pltpu.sync_copy(
                        rows[b],
                        out_hbm.at[pl.ds(base + j * _ROWS_PER_DMA,
                                         _ROWS_PER_DMA)])
            return 0

        lax.fori_loop(0, n_chunks, chunk, 0)

    return k(word_emb, tok_flat)


_EPS = 1e-12


def _ln_body(g_ref, seg_ref, pe_ref, te_ref, gamma_ref, beta_ref, o_ref):
    pe = pe_ref[...]
    te0 = te_ref[0, :]
    te1 = te_ref[1, :]
    seg = seg_ref[...]
    te = jnp.where(seg[..., None] == 0, te0[None, None, :], te1[None, None, :])
    x = g_ref[...] + te + pe[None, :, :]
    mu = jnp.mean(x, axis=-1, keepdims=True)
    xc = x - mu
    var = jnp.mean(xc * xc, axis=-1, keepdims=True)
    normed = xc * lax.rsqrt(var + _EPS)
    o_ref[...] = normed * gamma_ref[0, :] + beta_ref[0, :]


def _tc_ln(gathered, segment_ids, pe, te, gamma, beta, b_blk):
    B, S, D = gathered.shape
    grid = (B // b_blk,)
    return pl.pallas_call(
        _ln_body,
        grid=grid,
        in_specs=[
            pl.BlockSpec((b_blk, S, D), lambda i: (i, 0, 0)),
            pl.BlockSpec((b_blk, S), lambda i: (i, 0)),
            pl.BlockSpec((S, D), lambda i: (0, 0)),
            pl.BlockSpec((2, D), lambda i: (0, 0)),
            pl.BlockSpec((1, D), lambda i: (0, 0)),
            pl.BlockSpec((1, D), lambda i: (0, 0)),
        ],
        out_specs=pl.BlockSpec((b_blk, S, D), lambda i: (i, 0, 0)),
        out_shape=jax.ShapeDtypeStruct((B, S, D), jnp.float32),
    )(gathered, segment_ids, pe, te, gamma, beta)


def kernel(token_ids, segment_ids, word_emb, pos_emb, type_emb, gamma, beta):
    B, S = token_ids.shape
    D = word_emb.shape[1]
    N = B * S
    NW = 32
    n_chunks = N // (NW * _ROWS_PER_DMA)
    tok_flat = token_ids.reshape(N).astype(jnp.int32)
    gathered = _sc_gather(word_emb, tok_flat, n_chunks)
    out = _tc_ln(
        gathered.reshape(B, S, D),
        segment_ids.astype(jnp.int32),
        pos_emb[:S],
        type_emb,
        gamma.reshape(1, D),
        beta.reshape(1, D),
        b_blk=8,
    )
    return out


# speedup vs baseline: 4.9143x; 1.2679x over previous
import functools

import jax
import jax.numpy as jnp
from jax import lax
from jax.experimental import pallas as pl
from jax.experimental.pallas import tpu as pltpu
from jax.experimental.pallas import tpu_sc as plsc


_ROWS_PER_DMA = 128


def _sc_gather(word_emb, tok_flat, n_chunks):
    D = word_emb.shape[1]
    info = plsc.get_sparse_core_info()
    NC, NS = info.num_cores, info.num_subcores
    NW = NC * NS
    N = tok_flat.shape[0]
    per_w = N // NW
    mesh = plsc.VectorSubcoreMesh(core_axis_name="c", subcore_axis_name="s")

    @functools.partial(
        pl.kernel,
        out_type=jax.ShapeDtypeStruct((N, D), jnp.float32),
        mesh=mesh,
        scratch_types=[
            pltpu.VMEM((per_w,), jnp.int32),
            pltpu.VMEM((_ROWS_PER_DMA, D), jnp.float32),
            pltpu.VMEM((_ROWS_PER_DMA, D), jnp.float32),
            pltpu.SemaphoreType.DMA,
            pltpu.SemaphoreType.DMA,
        ],
    )
    def k(table_hbm, idx_hbm, out_hbm, idx_v, rows0, rows1, sem0, sem1):
        wid = lax.axis_index("s") * NC + lax.axis_index("c")
        base = wid * per_w
        pltpu.sync_copy(idx_hbm.at[pl.ds(base, per_w)], idx_v)
        rows = (rows0, rows1)
        sems = (sem0, sem1)
        pltpu.async_copy(
            table_hbm.at[idx_v.at[pl.ds(0, _ROWS_PER_DMA)]], rows0, sem0)

        def chunk(j, _):
            @pl.when(j + 1 < n_chunks)
            def _():
                for b in range(2):
                    @pl.when(lax.rem(j + 1, 2) == b)
                    def _():
                        pltpu.async_copy(
                            table_hbm.at[
                                idx_v.at[pl.ds((j + 1) * _ROWS_PER_DMA,
                                               _ROWS_PER_DMA)]],
                            rows[b], sems[b])

            for b in range(2):
                @pl.when(lax.rem(j, 2) == b)
                def _():
                    pltpu.make_async_copy(
                        table_hbm.at[
                            idx_v.at[pl.ds(j * _ROWS_PER_DMA,
                                           _ROWS_PER_DMA)]],
                        rows[b], sems[b]).wait()
                    pltpu.sync_copy(
                        rows[b],
                        out_hbm.at[pl.ds(base + j * _ROWS_PER_DMA,
                                         _ROWS_PER_DMA)])
            return 0

        lax.fori_loop(0, n_chunks, chunk, 0)

    return k(word_emb, tok_flat)


_EPS = 1e-12


def _ln_body(g_ref, seg_ref, pe_ref, te_ref, gamma_ref, beta_ref, o_ref):
    pe = pe_ref[...]
    te0 = te_ref[0, :]
    te1 = te_ref[1, :]
    seg = seg_ref[...]
    te = jnp.where(seg[..., None] == 0, te0[None, None, :], te1[None, None, :])
    x = g_ref[...] + te + pe[None, :, :]
    mu = jnp.mean(x, axis=-1, keepdims=True)
    xc = x - mu
    var = jnp.mean(xc * xc, axis=-1, keepdims=True)
    normed = xc * lax.rsqrt(var + _EPS)
    o_ref[...] = normed * gamma_ref[0, :] + beta_ref[0, :]


def _tc_ln(gathered, segment_ids, pe, te, gamma, beta, b_blk):
    B, S, D = gathered.shape
    grid = (B // b_blk,)
    return pl.pallas_call(
        _ln_body,
        grid=grid,
        in_specs=[
            pl.BlockSpec((b_blk, S, D), lambda i: (i, 0, 0)),
            pl.BlockSpec((b_blk, S), lambda i: (i, 0)),
            pl.BlockSpec((S, D), lambda i: (0, 0)),
            pl.BlockSpec((2, D), lambda i: (0, 0)),
            pl.BlockSpec((1, D), lambda i: (0, 0)),
            pl.BlockSpec((1, D), lambda i: (0, 0)),
        ],
        out_specs=pl.BlockSpec((b_blk, S, D), lambda i: (i, 0, 0)),
        out_shape=jax.ShapeDtypeStruct((B, S, D), jnp.float32),
    )(gathered, segment_ids, pe, te, gamma, beta)


def kernel(token_ids, segment_ids, word_emb, pos_emb, type_emb, gamma, beta):
    B, S = token_ids.shape
    D = word_emb.shape[1]
    N = B * S
    NW = 32
    n_chunks = N // (NW * _ROWS_PER_DMA)
    tok_flat = token_ids.reshape(N).astype(jnp.int32)
    gathered = _sc_gather(word_emb, tok_flat, n_chunks)
    out = _tc_ln(
        gathered.reshape(B, S, D),
        segment_ids.astype(jnp.int32),
        pos_emb[:S],
        type_emb,
        gamma.reshape(1, D),
        beta.reshape(1, D),
        b_blk=32,
    )
    return out


# speedup vs baseline: 5.0928x; 1.0363x over previous
import functools

import jax
import jax.numpy as jnp
from jax import lax
from jax.experimental import pallas as pl
from jax.experimental.pallas import tpu as pltpu
from jax.experimental.pallas import tpu_sc as plsc


_ROWS_PER_DMA = 128


def _sc_gather(word_emb, tok_flat, n_chunks):
    D = word_emb.shape[1]
    info = plsc.get_sparse_core_info()
    NC, NS = info.num_cores, info.num_subcores
    NW = NC * NS
    N = tok_flat.shape[0]
    per_w = N // NW
    mesh = plsc.VectorSubcoreMesh(core_axis_name="c", subcore_axis_name="s")

    @functools.partial(
        pl.kernel,
        out_type=jax.ShapeDtypeStruct((N, D), jnp.float32),
        mesh=mesh,
        scratch_types=[
            pltpu.VMEM((per_w,), jnp.int32),
            pltpu.VMEM((_ROWS_PER_DMA, D), jnp.float32),
            pltpu.VMEM((_ROWS_PER_DMA, D), jnp.float32),
            pltpu.SemaphoreType.DMA,
            pltpu.SemaphoreType.DMA,
        ],
    )
    def k(table_hbm, idx_hbm, out_hbm, idx_v, rows0, rows1, sem0, sem1):
        wid = lax.axis_index("s") * NC + lax.axis_index("c")
        base = wid * per_w
        pltpu.sync_copy(idx_hbm.at[pl.ds(base, per_w)], idx_v)
        rows = (rows0, rows1)
        sems = (sem0, sem1)
        pltpu.async_copy(
            table_hbm.at[idx_v.at[pl.ds(0, _ROWS_PER_DMA)]], rows0, sem0)

        def chunk(j, _):
            @pl.when(j + 1 < n_chunks)
            def _():
                for b in range(2):
                    @pl.when(lax.rem(j + 1, 2) == b)
                    def _():
                        pltpu.async_copy(
                            table_hbm.at[
                                idx_v.at[pl.ds((j + 1) * _ROWS_PER_DMA,
                                               _ROWS_PER_DMA)]],
                            rows[b], sems[b])

            for b in range(2):
                @pl.when(lax.rem(j, 2) == b)
                def _():
                    pltpu.make_async_copy(
                        table_hbm.at[
                            idx_v.at[pl.ds(j * _ROWS_PER_DMA,
                                           _ROWS_PER_DMA)]],
                        rows[b], sems[b]).wait()
                    pltpu.sync_copy(
                        rows[b],
                        out_hbm.at[pl.ds(base + j * _ROWS_PER_DMA,
                                         _ROWS_PER_DMA)])
            return 0

        lax.fori_loop(0, n_chunks, chunk, 0)

    return k(word_emb, tok_flat)


_EPS = 1e-12


def _ln_body(g_ref, seg_ref, pe_ref, te_ref, gamma_ref, beta_ref, o_ref):
    pe = pe_ref[...]
    te0 = te_ref[0, :]
    te1 = te_ref[1, :]
    seg = seg_ref[...]
    te = jnp.where(seg[..., None] == 0, te0[None, None, :], te1[None, None, :])
    x = g_ref[...] + te + pe[None, :, :]
    mu = jnp.mean(x, axis=-1, keepdims=True)
    xc = x - mu
    var = jnp.mean(xc * xc, axis=-1, keepdims=True)
    normed = xc * lax.rsqrt(var + _EPS)
    o_ref[...] = normed * gamma_ref[0, :] + beta_ref[0, :]


def _tc_ln(gathered, segment_ids, pe, te, gamma, beta, b_blk):
    B, S, D = gathered.shape
    grid = (B // b_blk,)
    return pl.pallas_call(
        _ln_body,
        grid=grid,
        in_specs=[
            pl.BlockSpec((b_blk, S, D), lambda i: (i, 0, 0)),
            pl.BlockSpec((b_blk, S), lambda i: (i, 0)),
            pl.BlockSpec((S, D), lambda i: (0, 0)),
            pl.BlockSpec((2, D), lambda i: (0, 0)),
            pl.BlockSpec((1, D), lambda i: (0, 0)),
            pl.BlockSpec((1, D), lambda i: (0, 0)),
        ],
        out_specs=pl.BlockSpec((b_blk, S, D), lambda i: (i, 0, 0)),
        out_shape=jax.ShapeDtypeStruct((B, S, D), jnp.float32),
    )(gathered, segment_ids, pe, te, gamma, beta)


def kernel(token_ids, segment_ids, word_emb, pos_emb, type_emb, gamma, beta):
    B, S = token_ids.shape
    D = word_emb.shape[1]
    N = B * S
    NW = 32
    n_chunks = N // (NW * _ROWS_PER_DMA)
    tok_flat = token_ids.reshape(N).astype(jnp.int32)
    gathered = _sc_gather(word_emb, tok_flat, n_chunks)
    out = _tc_ln(
        gathered.reshape(B, S, D),
        segment_ids.astype(jnp.int32),
        pos_emb[:S],
        type_emb,
        gamma.reshape(1, D),
        beta.reshape(1, D),
        b_blk=64,
    )
    return out
